# baseline (speedup 1.0000x reference)
import functools

import numpy as np
import jax
import jax.numpy as jnp
from jax import lax
from jax.experimental import pallas as pl
from jax.experimental.pallas import tpu as pltpu
from jax.experimental.pallas import tpu_sc as plsc

_LENGTH = 100
_DIM = 100
_PAD = 104
_BATCH = 4096
_NW = 32
_BPW = _BATCH // _NW
_ETILES = _PAD // 8
_PTILES = 25 - _ETILES
_NGB = 3
_NTB = 2


def _pe_full():
    pe = np.zeros((_LENGTH, _DIM))
    for pos in range(_LENGTH):
        for i in range(_DIM):
            pe[pos, i] = pos / np.power(10000, (i - i % 2) / _DIM)
    pe[:, 0::2] = np.sin(pe[:, 0::2])
    pe[:, 1::2] = np.cos(pe[:, 1::2])
    return pe.astype(np.float32)


def _pe_tiles():
    pe = _pe_full()
    t = pe[:, 4:]
    t = t.reshape(_LENGTH, _PTILES, 8, 1)
    return jnp.asarray(np.broadcast_to(t, (_LENGTH, _PTILES, 8, 128)).copy())


def _pe_head():
    pe = _pe_full()
    h = pe[:, :4].reshape(_LENGTH, 4, 1)
    return jnp.asarray(np.broadcast_to(h, (_LENGTH, 4, 16)).copy())


def _sc_lookup(idx_t, table, pe_tiles, pe_head):
    mesh = plsc.VectorSubcoreMesh(core_axis_name="c", subcore_axis_name="s")

    @functools.partial(
        pl.kernel,
        mesh=mesh,
        out_type=jax.ShapeDtypeStruct((_LENGTH, 25, _NW, 8, 128), jnp.float32),
        scratch_types=[
            pltpu.VMEM((_LENGTH, _BPW), jnp.int32),
            pltpu.VMEM((_NGB, _BPW, _PAD), jnp.float32),
            pltpu.VMEM((_NTB, _ETILES, 8, 128), jnp.float32),
            pltpu.VMEM((_LENGTH, 4, 16), jnp.float32),
            pltpu.VMEM_SHARED((_LENGTH // 2, _PTILES, 8, 128), jnp.float32),
            pltpu.SemaphoreType.DMA((_NGB,)),
            pltpu.SemaphoreType.DMA((_NTB,)),
            pltpu.SemaphoreType.DMA,
        ],
        compiler_params=pltpu.CompilerParams(
            use_tc_tiling_on_sc=False, needs_layout_passes=False
        ),
    )
    def body(idx_hbm, table_hbm, pet_hbm, peh_hbm, out_hbm,
             idx_v, gbuf, tbuf, pehv, shpe, sg, sw, sp):
        cid = lax.axis_index("c")
        sid = lax.axis_index("s")
        wid = sid * 2 + cid

        for k in range(4):
            ll = sid + 16 * k

            @pl.when(ll < _LENGTH // 2)
            def _load():
                pltpu.sync_copy(pet_hbm.at[cid * (_LENGTH // 2) + ll], shpe.at[ll])

        pltpu.sync_copy(idx_hbm.at[:, pl.ds(wid * _BPW, _BPW)], idx_v)
        pltpu.sync_copy(peh_hbm, pehv)
        plsc.subcore_barrier()

        lanes = lax.iota(jnp.int32, 16)
        rowv = [b0 * 16 + lanes for b0 in range(8)]

        def gather_into(l, b):
            pltpu.async_copy(table_hbm.at[idx_v.at[l]], gbuf.at[b], sg.at[b])

        def twrite(l, tb):
            return pltpu.make_async_copy(
                tbuf.at[tb],
                out_hbm.at[l, pl.ds(0, _ETILES), wid, :, :],
                sw.at[tb],
            )

        for k in range(_NGB):
            gather_into(k, k)

        def step(l, carry):
            b = lax.rem(l, _NGB)
            tb = lax.rem(l, _NTB)

            pltpu.make_async_copy(
                table_hbm.at[idx_v.at[l]], gbuf.at[b], sg.at[b]
            ).wait()

            @pl.when(l >= _NTB)
            def _drain():
                twrite(l - _NTB, tb).wait()

            for c0 in range(0, _DIM, 2):
                vs = []
                for c in (c0, c0 + 1):
                    colv = jnp.full((16,), c, jnp.int32)
                    vs += [
                        plsc.load_gather(gbuf.at[b], [rowv[b0], colv])
                        for b0 in range(8)
                    ]
                for k, c in enumerate((c0, c0 + 1)):
                    for b0 in range(8):
                        tbuf[tb, c // 8, c % 8, pl.ds(b0 * 16, 16)] = vs[8 * k + b0]

            for s2 in range(4):
                v = pehv[l, s2, :]
                for b0 in range(8):
                    tbuf[tb, _ETILES - 1, 4 + s2, pl.ds(b0 * 16, 16)] = v

            twrite(l, tb).start()
            flat = sid * _LENGTH + l
            ll = flat // 32
            t0p = lax.rem(flat, 32)
            pltpu.async_copy(
                shpe.at[ll],
                out_hbm.at[cid * (_LENGTH // 2) + ll,
                           pl.ds(_ETILES, _PTILES), t0p, :, :],
                sp,
            )

            @pl.when(l + _NGB < _LENGTH)
            def _fire():
                gather_into(l + _NGB, b)

            return carry

        lax.fori_loop(0, _LENGTH, step, 0)

        for k in range(_NTB):
            l = _LENGTH - _NTB + k
            twrite(l, l % _NTB).wait()

        def pdrain(l, carry):
            flat = sid * _LENGTH + l
            ll = flat // 32
            t0p = lax.rem(flat, 32)
            pltpu.make_async_copy(
                shpe.at[ll],
                out_hbm.at[cid * (_LENGTH // 2) + ll,
                           pl.ds(_ETILES, _PTILES), t0p, :, :],
                sp,
            ).wait()
            return carry

        lax.fori_loop(0, _LENGTH, pdrain, 0)

    return body(idx_t, table, pe_tiles, pe_head)


def kernel(inputs, embeddings):
    idx_t = inputs.T
    tpad = jnp.concatenate(
        [embeddings, jnp.zeros((embeddings.shape[0], _PAD - _DIM), embeddings.dtype)],
        axis=1,
    )
    out5 = _sc_lookup(idx_t, tpad, _pe_tiles(), _pe_head())
    t = jnp.transpose(out5, (2, 4, 0, 1, 3))
    return t.reshape(_BATCH, _LENGTH, 2 * _DIM)
